# baseline (speedup 1.0000x reference)
import jax
import jax.numpy as jnp
from jax import lax
from jax.experimental import pallas as pl
from jax.experimental.pallas import tpu as pltpu
from jax.experimental.pallas import tpu_sc as plsc

N = 4096
D = 128
DW = D // 4
NC, NS = 2, 16
NW = NC * NS
RPW = N // NW

N_MORPH = 160
N_TAG = 48
TAG_OFF = 168
IDX_W = 216
NPT = 64

SCALE = 64.0
W_MORPH = 1.0 / (3.0 * N_MORPH * SCALE)
W_TAG = 1.0 / (3.0 * N_TAG)
W_WORD = 1.0 / 3.0

BLK = 8
NBLK = RPW // BLK
RING = 4


def _sc_body(midx_hbm, tidx_hbm, widx_hbm, emb8_hbm, ptab_hbm, embf_hbm,
             lut_hbm, out_hbm, midxb, tidxb, widxb, ebuf, tbuf, wbuf, lut,
             ptl, mrow, oblk0, oblk1, sem_e, sem_t, sem_w, sem_o):
    wid = lax.axis_index("s") * NC + lax.axis_index("c")
    base = pl.multiple_of(wid * RPW, RPW)

    @pl.when(lax.axis_index("s") == 0)
    def _():
        pltpu.sync_copy(ptab_hbm, ptl)

    pltpu.sync_copy(midx_hbm.at[pl.ds(base, RPW)], midxb)
    pltpu.sync_copy(tidx_hbm.at[pl.ds(base, RPW)], tidxb)
    pltpu.sync_copy(widx_hbm.at[pl.ds(base, RPW)], widxb)
    pltpu.sync_copy(lut_hbm, lut)
    plsc.subcore_barrier()

    for k in range(RPW // 16):
        pltpu.async_copy(
            embf_hbm.at[widxb.at[pl.ds(16 * k, 16)]],
            wbuf.at[pl.ds(16 * k, 16)], sem_w)

    def fire(r, sl):
        pltpu.async_copy(
            emb8_hbm.at[midxb.at[r]], ebuf.at[sl], sem_e.at[sl])
        pltpu.async_copy(
            ptl.at[tidxb.at[r]], tbuf.at[sl], sem_t.at[sl])

    for r0 in range(RING - 1):
        fire(r0, r0)

    pltpu.make_async_copy(embf_hbm.at[pl.ds(0, RPW)], wbuf, sem_w).wait()

    zeros16 = jnp.zeros((16,), jnp.float32)
    mask_lo = jnp.int32(255)

    def blk2_body(rb2, _):
        for sb, oblk in ((0, oblk0), (1, oblk1)):
            rb = rb2 * 2 + sb

            @pl.when(rb2 >= 1)
            def _():
                pltpu.make_async_copy(
                    oblk, out_hbm.at[pl.ds(0, BLK)], sem_o.at[sb]).wait()

            for k in range(BLK):
                r = rb * BLK + k
                sl = k % RING

                @pl.when(r + RING - 1 < RPW)
                def _():
                    fire(r + RING - 1, (k + RING - 1) % RING)

                pltpu.make_async_copy(
                    emb8_hbm.at[pl.ds(0, N_MORPH)], ebuf.at[sl],
                    sem_e.at[sl]).wait()
                pltpu.make_async_copy(
                    ptab_hbm.at[pl.ds(0, N_TAG)], tbuf.at[sl],
                    sem_t.at[sl]).wait()

                def macc(j, carry):
                    acc = list(carry)
                    for v in range(2):
                        w = ebuf[sl, j, pl.ds(16 * v, 16)]
                        for kk in range(4):
                            b = lax.shift_right_logical(w, 8 * kk) & mask_lo
                            acc[4 * v + kk] = acc[4 * v + kk] + \
                                plsc.load_gather(lut, [b])
                    return tuple(acc)

                m = lax.fori_loop(0, N_MORPH, macc, (zeros16,) * 8, unroll=4)

                def tacc(j, carry):
                    return tuple(carry[c] + tbuf[sl, j, pl.ds(16 * c, 16)]
                                 for c in range(8))

                t = lax.fori_loop(0, N_TAG, tacc, (zeros16,) * 8, unroll=4)

                four_iota = lax.iota(jnp.int32, 16) * 4
                for v in range(2):
                    for kk in range(4):
                        plsc.store_scatter(
                            mrow, [four_iota + (64 * v + kk)],
                            m[4 * v + kk] * W_MORPH)
                for c in range(8):
                    wv = wbuf[r, pl.ds(16 * c, 16)]
                    oblk[k, pl.ds(16 * c, 16)] = (
                        mrow[pl.ds(16 * c, 16)] + t[c] * W_TAG + wv * W_WORD)

            start = pl.multiple_of(base + rb * BLK, BLK)
            pltpu.async_copy(
                oblk, out_hbm.at[pl.ds(start, BLK)], sem_o.at[sb])
        return 0

    lax.fori_loop(0, NBLK // 2, blk2_body, 0)

    pltpu.make_async_copy(
        oblk0, out_hbm.at[pl.ds(0, BLK)], sem_o.at[0]).wait()
    pltpu.make_async_copy(
        oblk1, out_hbm.at[pl.ds(0, BLK)], sem_o.at[1]).wait()


@jax.jit
def _run(midx, tidx, widx, emb8_i32, ptabf, embedding, lut_f32):
    mesh = plsc.VectorSubcoreMesh(
        core_axis_name="c", subcore_axis_name="s", num_cores=NC, num_subcores=NS)
    fn = pl.kernel(
        _sc_body,
        out_type=jax.ShapeDtypeStruct((N, D), jnp.float32),
        mesh=mesh,
        compiler_params=pltpu.CompilerParams(
            use_tc_tiling_on_sc=False, needs_layout_passes=False),
        scratch_types=[
            pltpu.VMEM((RPW, N_MORPH), jnp.int32),
            pltpu.VMEM((RPW, N_TAG), jnp.int32),
            pltpu.VMEM((RPW,), jnp.int32),
            pltpu.VMEM((RING, N_MORPH, DW), jnp.int32),
            pltpu.VMEM((RING, N_TAG, D), jnp.float32),
            pltpu.VMEM((RPW, D), jnp.float32),
            pltpu.VMEM((256,), jnp.float32),
            pltpu.VMEM_SHARED((NPT, D), jnp.float32),
            pltpu.VMEM((D,), jnp.float32),
            pltpu.VMEM((BLK, D), jnp.float32),
            pltpu.VMEM((BLK, D), jnp.float32),
            pltpu.SemaphoreType.DMA((RING,)),
            pltpu.SemaphoreType.DMA((RING,)),
            pltpu.SemaphoreType.DMA,
            pltpu.SemaphoreType.DMA((2,)),
        ],
    )
    return fn(midx, tidx, widx, emb8_i32, ptabf, embedding, lut_f32)


def _f8_words(x):
    y = jax.lax.bitcast_convert_type(x, jnp.int32)
    s = jnp.right_shift(y, 31) & 1
    a = (y & jnp.int32(0x7FFFFFFF)) + jnp.int32(0x80000)
    e = jnp.right_shift(a, 23) - 120
    m3 = jnp.right_shift(a, 20) & 7
    q = jnp.rint(jnp.abs(x) * 512.0).astype(jnp.int32)
    byte = jnp.where(e >= 1, jnp.left_shift(e, 3) | m3, q)
    byte = byte | jnp.left_shift(s, 7)
    b = byte.reshape(-1, DW, 4)
    return (b[:, :, 0] | jnp.left_shift(b[:, :, 1], 8)
            | jnp.left_shift(b[:, :, 2], 16) | jnp.left_shift(b[:, :, 3], 24))


def kernel(word_ids, morph_ids, embedding, postag_embedding):
    emb8_i32 = _f8_words(embedding * SCALE)
    lut_f32 = jax.lax.bitcast_convert_type(
        jnp.arange(256, dtype=jnp.uint8), jnp.float8_e4m3fn
    ).astype(jnp.float32)
    morph_flat = morph_ids[:, :, :-1, :].reshape(N, N_MORPH).astype(jnp.int32)
    tag_flat = morph_ids[:, :, :, -1].reshape(N, N_TAG).astype(jnp.int32)
    widx = word_ids.astype(jnp.int32)
    return _run(morph_flat, tag_flat, widx, emb8_i32, postag_embedding,
                embedding, lut_f32)


# speedup vs baseline: 2.2540x; 2.2540x over previous
import jax
import jax.numpy as jnp
from jax import lax
from jax.experimental import pallas as pl
from jax.experimental.pallas import tpu as pltpu
from jax.experimental.pallas import tpu_sc as plsc

N = 4096
D = 128
DW = D // 4
NC, NS = 2, 16
NW = NC * NS
RPW = N // NW

N_MORPH = 160
N_TAG = 48
TAG_OFF = 168
IDX_W = 216
NPT = 64

SCALE = 64.0
W_MORPH = 1.0 / (3.0 * N_MORPH * SCALE)
W_TAG = 1.0 / (3.0 * N_TAG)
W_WORD = 1.0 / 3.0

BLK = 8
NBLK = RPW // BLK
RING = 4


def _sc_body(midx_hbm, tidx_hbm, widx_hbm, emb8_hbm, ptab_hbm, embf_hbm,
             lut_hbm, out_hbm, midxb, tidxb, widxb, ebuf, tbuf, wbuf, lut,
             ptl, mrow, oblk0, oblk1, sem_e, sem_t, sem_w, sem_o):
    wid = lax.axis_index("s") * NC + lax.axis_index("c")
    base = pl.multiple_of(wid * RPW, RPW)

    @pl.when(lax.axis_index("s") == 0)
    def _():
        pltpu.sync_copy(ptab_hbm, ptl)

    pltpu.sync_copy(midx_hbm.at[pl.ds(base, RPW)], midxb)
    pltpu.sync_copy(tidx_hbm.at[pl.ds(base, RPW)], tidxb)
    pltpu.sync_copy(widx_hbm.at[pl.ds(base, RPW)], widxb)
    pltpu.sync_copy(lut_hbm, lut)
    plsc.subcore_barrier()

    for k in range(RPW // 16):
        pltpu.async_copy(
            embf_hbm.at[widxb.at[pl.ds(16 * k, 16)]],
            wbuf.at[pl.ds(16 * k, 16)], sem_w)

    def fire(r, sl):
        pltpu.async_copy(
            emb8_hbm.at[midxb.at[r]], ebuf.at[sl], sem_e.at[sl])
        pltpu.async_copy(
            ptl.at[tidxb.at[r]], tbuf.at[sl], sem_t.at[sl])

    for r0 in range(RING - 1):
        fire(r0, r0)

    pltpu.make_async_copy(embf_hbm.at[pl.ds(0, RPW)], wbuf, sem_w).wait()

    zeros16 = jnp.zeros((16,), jnp.float32)
    mask_lo = jnp.int32(255)

    def blk2_body(rb2, _):
        for sb, oblk in ((0, oblk0), (1, oblk1)):
            rb = rb2 * 2 + sb

            @pl.when(rb2 >= 1)
            def _():
                pltpu.make_async_copy(
                    oblk, out_hbm.at[pl.ds(0, BLK)], sem_o.at[sb]).wait()

            for k in range(BLK):
                r = rb * BLK + k
                sl = k % RING

                @pl.when(r + RING - 1 < RPW)
                def _():
                    fire(r + RING - 1, (k + RING - 1) % RING)

                pltpu.make_async_copy(
                    emb8_hbm.at[pl.ds(0, N_MORPH)], ebuf.at[sl],
                    sem_e.at[sl]).wait()
                pltpu.make_async_copy(
                    ptab_hbm.at[pl.ds(0, N_TAG)], tbuf.at[sl],
                    sem_t.at[sl]).wait()

                def macc(j, carry):
                    acc = list(carry)
                    for v in range(2):
                        w8 = ebuf[sl, j, pl.ds(64 * v, 64)]
                        w = plsc.bitcast(w8, jnp.int32)
                        for kk in range(4):
                            b = lax.shift_right_logical(w, 8 * kk) & mask_lo
                            acc[4 * v + kk] = acc[4 * v + kk] + \
                                plsc.load_gather(lut, [b])
                    return tuple(acc)

                m = lax.fori_loop(0, N_MORPH, macc, (zeros16,) * 8, unroll=4)

                def tacc(j, carry):
                    return tuple(carry[c] + tbuf[sl, j, pl.ds(16 * c, 16)]
                                 for c in range(8))

                t = lax.fori_loop(0, N_TAG, tacc, (zeros16,) * 8, unroll=4)

                four_iota = lax.iota(jnp.int32, 16) * 4
                for v in range(2):
                    for kk in range(4):
                        plsc.store_scatter(
                            mrow, [four_iota + (64 * v + kk)],
                            m[4 * v + kk] * W_MORPH)
                for c in range(8):
                    wv = wbuf[r, pl.ds(16 * c, 16)]
                    oblk[k, pl.ds(16 * c, 16)] = (
                        mrow[pl.ds(16 * c, 16)] + t[c] * W_TAG + wv * W_WORD)

            start = pl.multiple_of(base + rb * BLK, BLK)
            pltpu.async_copy(
                oblk, out_hbm.at[pl.ds(start, BLK)], sem_o.at[sb])
        return 0

    lax.fori_loop(0, NBLK // 2, blk2_body, 0)

    pltpu.make_async_copy(
        oblk0, out_hbm.at[pl.ds(0, BLK)], sem_o.at[0]).wait()
    pltpu.make_async_copy(
        oblk1, out_hbm.at[pl.ds(0, BLK)], sem_o.at[1]).wait()


@jax.jit
def _run(midx, tidx, widx, emb8, ptabf, embedding, lut_f32):
    mesh = plsc.VectorSubcoreMesh(
        core_axis_name="c", subcore_axis_name="s", num_cores=NC, num_subcores=NS)
    fn = pl.kernel(
        _sc_body,
        out_type=jax.ShapeDtypeStruct((N, D), jnp.float32),
        mesh=mesh,
        compiler_params=pltpu.CompilerParams(
            use_tc_tiling_on_sc=False, needs_layout_passes=False),
        scratch_types=[
            pltpu.VMEM((RPW, N_MORPH), jnp.int32),
            pltpu.VMEM((RPW, N_TAG), jnp.int32),
            pltpu.VMEM((RPW,), jnp.int32),
            pltpu.VMEM((RING, N_MORPH, D), jnp.float8_e4m3fn),
            pltpu.VMEM((RING, N_TAG, D), jnp.float32),
            pltpu.VMEM((RPW, D), jnp.float32),
            pltpu.VMEM((256,), jnp.float32),
            pltpu.VMEM_SHARED((NPT, D), jnp.float32),
            pltpu.VMEM((D,), jnp.float32),
            pltpu.VMEM((BLK, D), jnp.float32),
            pltpu.VMEM((BLK, D), jnp.float32),
            pltpu.SemaphoreType.DMA((RING,)),
            pltpu.SemaphoreType.DMA((RING,)),
            pltpu.SemaphoreType.DMA,
            pltpu.SemaphoreType.DMA((2,)),
        ],
    )
    return fn(midx, tidx, widx, emb8, ptabf, embedding, lut_f32)


def kernel(word_ids, morph_ids, embedding, postag_embedding):
    emb8 = (embedding * SCALE).astype(jnp.float8_e4m3fn)
    lut_f32 = jax.lax.bitcast_convert_type(
        jnp.arange(256, dtype=jnp.uint8), jnp.float8_e4m3fn
    ).astype(jnp.float32)
    morph_flat = morph_ids[:, :, :-1, :].reshape(N, N_MORPH).astype(jnp.int32)
    tag_flat = morph_ids[:, :, :, -1].reshape(N, N_TAG).astype(jnp.int32)
    widx = word_ids.astype(jnp.int32)
    return _run(morph_flat, tag_flat, widx, emb8, postag_embedding,
                embedding, lut_f32)
